# baseline (speedup 1.0000x reference)
import jax
import jax.numpy as jnp
from jax.experimental import pallas as pl
from jax.experimental.pallas import tpu as pltpu

LAT_H = 48
LAT_W = 80
POOL_H = 8
POOL_W = 16
VIS_LEN = 30720
N_HEADS = 12
HEAD_DIM = 64
SPARSITY = 0.9

_PARTS = VIS_LEN // (LAT_W * POOL_H)
_B = POOL_H
_C = LAT_W // POOL_W
_CHUNK = POOL_W
_PART = _B * _C * _CHUNK
_S = _PARTS * _C
_N = _S * _S
_KCNT = int((1.0 - (1.0 - SPARSITY)) * _N)


def _permute_pool_body(x_ref, xr_ref, xp_ref):
    x = x_ref[...].reshape(_B, _C, _CHUNK, N_HEADS, HEAD_DIM)
    y = x.transpose(1, 0, 2, 3, 4)
    xr_ref[...] = y.reshape(_PART, N_HEADS, HEAD_DIM)
    for c in range(_C):
        xp_ref[pl.ds(c, 1)] = jnp.sum(y[c], axis=(0, 1))[None]


def _permute_body(x_ref, xr_ref):
    x = x_ref[...].reshape(_B, _C, _CHUNK, N_HEADS, HEAD_DIM)
    xr_ref[...] = x.transpose(1, 0, 2, 3, 4).reshape(_PART, N_HEADS, HEAD_DIM)


def _attn_mask_body(qp_ref, kp_ref, m_ref):
    qh = qp_ref[0]
    kh = kp_ref[0]
    s = jax.lax.dot_general(qh, kh, (((1,), (1,)), ((), ())),
                            preferred_element_type=jnp.float32)
    s = s * jnp.float32(2.0 ** -17)
    mx = jnp.max(s, axis=-1, keepdims=True)
    e = jnp.exp(s - mx)
    attn = e / jnp.sum(e, axis=-1, keepdims=True)
    bits = jax.lax.bitcast_convert_type(attn, jnp.int32)

    def body(i, ans):
        bit = jnp.int32(30) - i
        cand = ans | jax.lax.shift_left(jnp.int32(1), bit)
        cnt = jnp.sum((bits < cand).astype(jnp.int32))
        return jnp.where(cnt < _KCNT, cand, ans)

    ans = jax.lax.fori_loop(0, 31, body, jnp.int32(0))
    thr = jax.lax.bitcast_convert_type(ans, jnp.float32)
    m_ref[0] = (attn >= thr).astype(jnp.int8)


def kernel(q, k, v, cu_seqlens_q, cu_seqlens_kv, max_seqlen_q, max_seqlen_kv):
    L, H, D = q.shape

    part_spec = pl.BlockSpec((_PART, H, D), lambda p: (p, 0, 0))
    pool_spec = pl.BlockSpec((_C, H, D), lambda p: (p, 0, 0))

    row_t = jax.ShapeDtypeStruct((L, H, D), jnp.float32)
    pool_t = jax.ShapeDtypeStruct((_S, H, D), jnp.float32)
    permute_pool = pl.pallas_call(
        _permute_pool_body,
        grid=(_PARTS,),
        in_specs=[part_spec],
        out_specs=[part_spec, pool_spec],
        out_shape=[row_t, pool_t],
    )
    permute = pl.pallas_call(
        _permute_body,
        grid=(_PARTS,),
        in_specs=[part_spec],
        out_specs=part_spec,
        out_shape=row_t,
    )
    q_r, qp = permute_pool(q)
    k_r, kp = permute_pool(k)
    v_r = permute(v)

    qp_h = qp.transpose(1, 0, 2)
    kp_h = kp.transpose(1, 0, 2)

    head_spec = pl.BlockSpec((1, _S, D), lambda h: (h, 0, 0))
    mask_i8 = pl.pallas_call(
        _attn_mask_body,
        grid=(H,),
        in_specs=[head_spec, head_spec],
        out_specs=pl.BlockSpec((1, _S, _S), lambda h: (h, 0, 0)),
        out_shape=jax.ShapeDtypeStruct((H, _S, _S), jnp.int8),
    )(qp_h, kp_h)

    mask = mask_i8.astype(bool).reshape(1, H, _S, _S)
    return (q_r, k_r, v_r, mask)


# speedup vs baseline: 1.9014x; 1.9014x over previous
import jax
import jax.numpy as jnp
from jax.experimental import pallas as pl
from jax.experimental.pallas import tpu as pltpu

LAT_H = 48
LAT_W = 80
POOL_H = 8
POOL_W = 16
VIS_LEN = 30720
N_HEADS = 12
HEAD_DIM = 64
SPARSITY = 0.9

_PARTS = VIS_LEN // (LAT_W * POOL_H)
_B = POOL_H
_C = LAT_W // POOL_W
_CHUNK = POOL_W
_S = _PARTS * _C
_N = _S * _S
_KCNT = int((1.0 - (1.0 - SPARSITY)) * _N)


def _permute_pool_body(x_ref, xr_ref, xp_ref, sem):
    p = pl.program_id(0)
    copies = []
    for c in range(_C):
        cp = pltpu.make_async_copy(
            x_ref.at[:, pl.ds(c * _CHUNK, _CHUNK)],
            xr_ref.at[pl.ds(p * _B * _C + c * _B, _B)],
            sem)
        cp.start()
        copies.append(cp)
    for c in range(_C):
        sl = pl.ds(c * _CHUNK, _CHUNK)
        xp_ref[pl.ds(c, 1)] = jnp.sum(x_ref[:, sl], axis=(0, 1))[None]
    for cp in copies:
        cp.wait()


def _permute_body(x_ref, xr_ref, sem):
    p = pl.program_id(0)
    copies = []
    for c in range(_C):
        cp = pltpu.make_async_copy(
            x_ref.at[:, pl.ds(c * _CHUNK, _CHUNK)],
            xr_ref.at[pl.ds(p * _B * _C + c * _B, _B)],
            sem)
        cp.start()
        copies.append(cp)
    for cp in copies:
        cp.wait()


def _attn_mask_body(qp_ref, kp_ref, m_ref):
    qh = qp_ref[0]
    kh = kp_ref[0]
    s = jax.lax.dot_general(qh, kh, (((1,), (1,)), ((), ())),
                            preferred_element_type=jnp.float32)
    s = s * jnp.float32(2.0 ** -17)
    mx = jnp.max(s, axis=-1, keepdims=True)
    e = jnp.exp(s - mx)
    attn = e / jnp.sum(e, axis=-1, keepdims=True)
    bits = jax.lax.bitcast_convert_type(attn, jnp.int32)

    def body(i, ans):
        bit = jnp.int32(30) - i
        cand = ans | jax.lax.shift_left(jnp.int32(1), bit)
        cnt = jnp.sum((bits < cand).astype(jnp.int32))
        return jnp.where(cnt < _KCNT, cand, ans)

    ans = jax.lax.fori_loop(0, 31, body, jnp.int32(0))
    thr = jax.lax.bitcast_convert_type(ans, jnp.float32)
    m_ref[0] = (attn >= thr).astype(jnp.int8)


def kernel(q, k, v, cu_seqlens_q, cu_seqlens_kv, max_seqlen_q, max_seqlen_kv):
    L, H, D = q.shape

    nrow_in = L // LAT_W
    q4 = q.reshape(nrow_in, LAT_W, H, D)
    k4 = k.reshape(nrow_in, LAT_W, H, D)
    v4 = v.reshape(nrow_in, LAT_W, H, D)

    in_spec = pl.BlockSpec((_B, LAT_W, H, D), lambda p: (p, 0, 0, 0))
    any_spec = pl.BlockSpec(memory_space=pl.ANY)
    pool_spec = pl.BlockSpec((_C, H, D), lambda p: (p, 0, 0))

    out4_t = jax.ShapeDtypeStruct((L // _CHUNK, _CHUNK, H, D), jnp.float32)
    pool_t = jax.ShapeDtypeStruct((_S, H, D), jnp.float32)
    permute_pool = pl.pallas_call(
        _permute_pool_body,
        grid=(_PARTS,),
        in_specs=[in_spec],
        out_specs=[any_spec, pool_spec],
        out_shape=[out4_t, pool_t],
        scratch_shapes=[pltpu.SemaphoreType.DMA],
    )
    permute = pl.pallas_call(
        _permute_body,
        grid=(_PARTS,),
        in_specs=[in_spec],
        out_specs=[any_spec],
        out_shape=[out4_t],
        scratch_shapes=[pltpu.SemaphoreType.DMA],
    )
    qr4, qp = permute_pool(q4)
    kr4, kp = permute_pool(k4)
    (vr4,) = permute(v4)

    q_r = qr4.reshape(L, H, D)
    k_r = kr4.reshape(L, H, D)
    v_r = vr4.reshape(L, H, D)

    qp_h = qp.transpose(1, 0, 2)
    kp_h = kp.transpose(1, 0, 2)

    head_spec = pl.BlockSpec((1, _S, D), lambda h: (h, 0, 0))
    mask_i8 = pl.pallas_call(
        _attn_mask_body,
        grid=(H,),
        in_specs=[head_spec, head_spec],
        out_specs=pl.BlockSpec((1, _S, _S), lambda h: (h, 0, 0)),
        out_shape=jax.ShapeDtypeStruct((H, _S, _S), jnp.int8),
    )(qp_h, kp_h)

    mask = mask_i8.astype(bool).reshape(1, H, _S, _S)
    return (q_r, k_r, v_r, mask)
